# baseline (speedup 1.0000x reference)
import jax
import jax.numpy as jnp
from jax import lax
from jax.experimental import pallas as pl
from jax.experimental.pallas import tpu as pltpu
from jax.experimental.pallas import tpu_sc as plsc

_B, _T, _D = 16, 576, 64
_K = 1024
_N = _B * _T
_TILE = 1152
_BPT = _TILE // _T
_NH = _N // 2
_HTILES = _NH // _TILE
_COMMITMENT_COST = 0.25
_ENTROPY_LOSS_RATIO = 0.1

_NW = 32
_RPW = _NH // _NW
_CHUNK = 72
_NCHUNK = _RPW // _CHUNK
_WPT = _TILE // _RPW
_DPAD = 128


def _tc_common(xt_ref, cbt_ref, idx_ref, idx3_ref, i):
    cbt = cbt_ref[...]
    mms = []
    x2s = []
    for b in range(_BPT):
        xb = xt_ref[b]
        mms.append(lax.dot_general(xb, cbt, (((0,), (0,)), ((), ())),
                                   preferred_element_type=jnp.float32))
        x2s.append(jnp.sum(xb * xb, axis=0))
    mm = jnp.concatenate(mms, axis=0)
    x2 = jnp.concatenate(x2s, axis=0).reshape(_TILE, 1)
    c2 = jnp.sum(cbt * cbt, axis=0)[None, :]
    d2 = jnp.maximum(x2 + c2 - 2.0 * mm, 0.0)
    d = jnp.sqrt(d2)

    dmin = jnp.min(d, axis=1, keepdims=True)
    kio = lax.broadcasted_iota(jnp.int32, (_TILE, _K), 1)
    idx = jnp.min(jnp.where(d == dmin, kio, _K), axis=1)
    idx_ref[pl.ds(i * _TILE, _TILE)] = idx
    idx3_ref[...] = idx.reshape(_WPT, _NCHUNK, _CHUNK)

    sh = dmin - d
    p = jnp.exp(sh)
    s = jnp.sum(p, axis=1, keepdims=True)
    t = jnp.sum(p * sh, axis=1, keepdims=True)
    samp = jnp.sum(t / s - jnp.log(s))
    sq = jnp.sum(dmin * dmin)
    pcol = jnp.sum(p * (1.0 / s), axis=0, keepdims=True)
    return cbt, pcol, sq, samp


def _tc_a_body(xt_ref, cbt_ref, idx_ref, idx3_ref, acc_ref, cbp_ref,
               accp_ref, accs_ref):
    i = pl.program_id(0)
    cbt, pcol, sq, samp = _tc_common(xt_ref, cbt_ref, idx_ref, idx3_ref, i)

    @pl.when(i == 0)
    def _():
        cbp_ref[:, :_D] = lax.transpose(cbt, (1, 0))
        cbp_ref[:, _D:] = jnp.zeros((_K, _DPAD - _D), jnp.float32)
        accp_ref[...] = jnp.zeros_like(accp_ref)
        accs_ref[0] = 0.0
        accs_ref[1] = 0.0

    accp_ref[...] += pcol
    accs_ref[0] += sq
    accs_ref[1] += samp

    @pl.when(i == _HTILES - 1)
    def _():
        acc_ref[0:1, :] = accp_ref[...]
        lane = lax.broadcasted_iota(jnp.int32, (1, _K), 1)
        row = jnp.where(lane == 0, accs_ref[0],
                        jnp.where(lane == 1, accs_ref[1], 0.0))
        acc_ref[1:2, :] = row


def _tc_b_body(xt_ref, cbt_ref, acc_ref, idx_ref, idx3_ref, loss_ref,
               accp_ref, accs_ref):
    i = pl.program_id(0)
    _, pcol, sq, samp = _tc_common(xt_ref, cbt_ref, idx_ref, idx3_ref, i)

    @pl.when(i == 0)
    def _():
        accp_ref[...] = acc_ref[0:1, :]
        lane = lax.broadcasted_iota(jnp.int32, (1, _K), 1)
        arow = acc_ref[1:2, :]
        accs_ref[0] = jnp.sum(jnp.where(lane == 0, arow, 0.0))
        accs_ref[1] = jnp.sum(jnp.where(lane == 1, arow, 0.0))

    accp_ref[...] += pcol
    accs_ref[0] += sq
    accs_ref[1] += samp

    @pl.when(i == _HTILES - 1)
    def _():
        avgp = accp_ref[...] / _N
        avg_ent = -jnp.sum(avgp * jnp.log(avgp + 1e-5))
        sample_ent = -(accs_ref[1] / _N)
        latent = (1.0 + _COMMITMENT_COST) * (accs_ref[0] / (_N * _D))
        loss = latent + _ENTROPY_LOSS_RATIO * (sample_ent - avg_ent)
        loss_ref[...] = jnp.full((1, 128), loss, jnp.float32)


_x_spec = pl.BlockSpec((_BPT, _D, _T), lambda i: (i, 0, 0))
_cb_spec = pl.BlockSpec((_D, _K), lambda i: (0, 0))
_idx_spec = pl.BlockSpec((_NH,), lambda i: (0,))
_idx3_spec = pl.BlockSpec((_WPT, _NCHUNK, _CHUNK), lambda i: (i, 0, 0))
_acc_spec = pl.BlockSpec((2, _K), lambda i: (0, 0))
_scratch = [
    pltpu.VMEM((1, _K), jnp.float32),
    pltpu.SMEM((2,), jnp.float32),
]

_vq_tc_a = pl.pallas_call(
    _tc_a_body,
    grid=(_HTILES,),
    in_specs=[_x_spec, _cb_spec],
    out_specs=[
        _idx_spec,
        _idx3_spec,
        _acc_spec,
        pl.BlockSpec((_K, _DPAD), lambda i: (0, 0)),
    ],
    out_shape=[
        jax.ShapeDtypeStruct((_NH,), jnp.int32),
        jax.ShapeDtypeStruct((_NW, _NCHUNK, _CHUNK), jnp.int32),
        jax.ShapeDtypeStruct((2, _K), jnp.float32),
        jax.ShapeDtypeStruct((_K, _DPAD), jnp.float32),
    ],
    scratch_shapes=_scratch,
)

_vq_tc_b = pl.pallas_call(
    _tc_b_body,
    grid=(_HTILES,),
    in_specs=[_x_spec, _cb_spec, _acc_spec],
    out_specs=[
        _idx_spec,
        _idx3_spec,
        pl.BlockSpec((1, 128), lambda i: (0, 0)),
    ],
    out_shape=[
        jax.ShapeDtypeStruct((_NH,), jnp.int32),
        jax.ShapeDtypeStruct((_NW, _NCHUNK, _CHUNK), jnp.int32),
        jax.ShapeDtypeStruct((1, 128), jnp.float32),
    ],
    scratch_shapes=_scratch,
)


def _sc_gather_body(cb_hbm, idx_hbm, out_hbm, idx_v, rows_v, sem):
    c = lax.axis_index("c")
    s = lax.axis_index("s")
    wid = s * 2 + c
    base = wid * _RPW
    pltpu.sync_copy(idx_hbm.at[wid], idx_v)
    for j in range(_NCHUNK):
        pltpu.async_copy(cb_hbm.at[idx_v.at[j]], rows_v, sem).wait()
        pltpu.sync_copy(rows_v,
                        out_hbm.at[pl.ds(base + j * _CHUNK, _CHUNK)])


_sc_gather_cache = []


def _sc_gather(cb_pad, idx3):
    if not _sc_gather_cache:
        _sc_gather_cache.append(pl.kernel(
            _sc_gather_body,
            out_type=jax.ShapeDtypeStruct((_NH, _DPAD), jnp.float32),
            mesh=plsc.VectorSubcoreMesh(core_axis_name="c",
                                        subcore_axis_name="s"),
            scratch_types=[
                pltpu.VMEM((_NCHUNK, _CHUNK), jnp.int32),
                pltpu.VMEM((_CHUNK, _DPAD), jnp.float32),
                pltpu.SemaphoreType.DMA,
            ],
        ))
    return _sc_gather_cache[0](cb_pad, idx3)


def kernel(x, codebook):
    xt = jnp.swapaxes(x, 1, 2)
    cbt = codebook.T
    xt_a = xt[:_B // 2]
    xt_b = xt[_B // 2:]
    idx_a, idx3_a, acc, cb_pad = _vq_tc_a(xt_a, cbt)
    q_a = _sc_gather(cb_pad, idx3_a)
    idx_b, idx3_b, loss_out = _vq_tc_b(xt_b, cbt, acc)
    q_b = _sc_gather(cb_pad, idx3_b)
    quantized = jnp.concatenate([q_a[:, :_D], q_b[:, :_D]], axis=0)
    idx = jnp.concatenate([idx_a, idx_b])
    return quantized.reshape(x.shape), loss_out[0, 0], idx


# speedup vs baseline: 1.1746x; 1.1746x over previous
import jax
import jax.numpy as jnp
from jax import lax
from jax.experimental import pallas as pl
from jax.experimental.pallas import tpu as pltpu
from jax.experimental.pallas import tpu_sc as plsc

_B, _T, _D = 16, 576, 64
_K = 1024
_N = _B * _T
_TILE = 1152
_BPT = _TILE // _T
_NTILES = _N // _TILE
_COMMITMENT_COST = 0.25
_ENTROPY_LOSS_RATIO = 0.1

_NW = 32
_ROWS_PER_W = _N // _NW
_CHUNK = 96
_NCHUNK = _ROWS_PER_W // _CHUNK
_WPT = _TILE // _ROWS_PER_W
_DPAD = 128


def _vq_tc_body(xt_ref, cbt_ref, idx_ref, idx3_ref, loss_ref, cbp_ref,
                accp_ref, accs_ref):
    i = pl.program_id(0)
    cbt = cbt_ref[...]
    mms = []
    x2s = []
    for b in range(_BPT):
        xb = xt_ref[b]
        mms.append(lax.dot_general(xb, cbt, (((0,), (0,)), ((), ())),
                                   preferred_element_type=jnp.float32))
        x2s.append(jnp.sum(xb * xb, axis=0))
    mm = jnp.concatenate(mms, axis=0)
    x2 = jnp.concatenate(x2s, axis=0).reshape(_TILE, 1)
    c2 = jnp.sum(cbt * cbt, axis=0)[None, :]
    d2 = jnp.maximum(x2 + c2 - 2.0 * mm, 0.0)
    d = jnp.sqrt(d2)

    dmin = jnp.min(d, axis=1, keepdims=True)
    kio = lax.broadcasted_iota(jnp.int32, (_TILE, _K), 1)
    idx = jnp.min(jnp.where(d == dmin, kio, _K), axis=1)
    idx_ref[pl.ds(i * _TILE, _TILE)] = idx
    idx3_ref[...] = idx.reshape(_WPT, _NCHUNK, _CHUNK)

    sh = dmin - d
    p = jnp.exp(sh)
    s = jnp.sum(p, axis=1, keepdims=True)
    t = jnp.sum(p * sh, axis=1, keepdims=True)
    samp = jnp.sum(t / s - jnp.log(s))
    sq = jnp.sum(dmin * dmin)
    pcol = jnp.sum(p * (1.0 / s), axis=0, keepdims=True)

    @pl.when(i == 0)
    def _():
        cbp_ref[:, :_D] = lax.transpose(cbt, (1, 0))
        cbp_ref[:, _D:] = jnp.zeros((_K, _DPAD - _D), jnp.float32)
        accp_ref[...] = jnp.zeros_like(accp_ref)
        accs_ref[0] = 0.0
        accs_ref[1] = 0.0

    accp_ref[...] += pcol
    accs_ref[0] += sq
    accs_ref[1] += samp

    @pl.when(i == _NTILES - 1)
    def _():
        avgp = accp_ref[...] / _N
        avg_ent = -jnp.sum(avgp * jnp.log(avgp + 1e-5))
        sample_ent = -(accs_ref[1] / _N)
        latent = (1.0 + _COMMITMENT_COST) * (accs_ref[0] / (_N * _D))
        loss = latent + _ENTROPY_LOSS_RATIO * (sample_ent - avg_ent)
        loss_ref[...] = jnp.full((1, 128), loss, jnp.float32)


_vq_tc = pl.pallas_call(
    _vq_tc_body,
    grid=(_NTILES,),
    in_specs=[
        pl.BlockSpec((_BPT, _D, _T), lambda i: (i, 0, 0)),
        pl.BlockSpec((_D, _K), lambda i: (0, 0)),
    ],
    out_specs=[
        pl.BlockSpec((_N,), lambda i: (0,)),
        pl.BlockSpec((_WPT, _NCHUNK, _CHUNK), lambda i: (i, 0, 0)),
        pl.BlockSpec((1, 128), lambda i: (0, 0)),
        pl.BlockSpec((_K, _DPAD), lambda i: (0, 0)),
    ],
    out_shape=[
        jax.ShapeDtypeStruct((_N,), jnp.int32),
        jax.ShapeDtypeStruct((_NW, _NCHUNK, _CHUNK), jnp.int32),
        jax.ShapeDtypeStruct((1, 128), jnp.float32),
        jax.ShapeDtypeStruct((_K, _DPAD), jnp.float32),
    ],
    scratch_shapes=[
        pltpu.VMEM((1, _K), jnp.float32),
        pltpu.SMEM((2,), jnp.float32),
    ],
)


def _sc_gather_body(cb_hbm, idx_hbm, out_hbm, idx_v, rows_v,
                    gsem0, gsem1, gsem2, ssem):
    c = lax.axis_index("c")
    s = lax.axis_index("s")
    wid = s * 2 + c
    base = wid * _ROWS_PER_W
    gsems = [gsem0, gsem1, gsem2]
    pltpu.sync_copy(idx_hbm.at[wid], idx_v)
    gathers = [
        pltpu.async_copy(cb_hbm.at[idx_v.at[j]], rows_v.at[j], gsems[j])
        for j in range(_NCHUNK)
    ]
    scatters = []
    for j in range(_NCHUNK):
        gathers[j].wait()
        scatters.append(pltpu.async_copy(
            rows_v.at[j], out_hbm.at[pl.ds(base + j * _CHUNK, _CHUNK)],
            ssem))
    for sc in scatters:
        sc.wait()


_sc_gather_cache = []


def _sc_gather(cb_pad, idx3):
    if not _sc_gather_cache:
        _sc_gather_cache.append(pl.kernel(
            _sc_gather_body,
            out_type=jax.ShapeDtypeStruct((_N, _DPAD), jnp.float32),
            mesh=plsc.VectorSubcoreMesh(core_axis_name="c",
                                        subcore_axis_name="s"),
            scratch_types=[
                pltpu.VMEM((_NCHUNK, _CHUNK), jnp.int32),
                pltpu.VMEM((_NCHUNK, _CHUNK, _DPAD), jnp.float32),
                pltpu.SemaphoreType.DMA,
                pltpu.SemaphoreType.DMA,
                pltpu.SemaphoreType.DMA,
                pltpu.SemaphoreType.DMA,
            ],
        ))
    return _sc_gather_cache[0](cb_pad, idx3)


def kernel(x, codebook):
    xt = jnp.swapaxes(x, 1, 2)
    cbt = codebook.T
    idx, idx3, loss_out, cb_pad = _vq_tc(xt, cbt)
    quantized = _sc_gather(cb_pad, idx3)
    return quantized[:, :_D].reshape(x.shape), loss_out[0, 0], idx


# speedup vs baseline: 1.1896x; 1.0128x over previous
import jax
import jax.numpy as jnp
from jax import lax
from jax.experimental import pallas as pl
from jax.experimental.pallas import tpu as pltpu
from jax.experimental.pallas import tpu_sc as plsc

_B, _T, _D = 16, 576, 64
_K = 1024
_N = _B * _T
_TILE = 2304
_BPT = _TILE // _T
_NTILES = _N // _TILE
_COMMITMENT_COST = 0.25
_ENTROPY_LOSS_RATIO = 0.1

_NW = 32
_ROWS_PER_W = _N // _NW
_CHUNK = 96
_NCHUNK = _ROWS_PER_W // _CHUNK
_WPT = _TILE // _ROWS_PER_W
_DPAD = 128


def _vq_tc_body(xt_ref, cbt_ref, idx_ref, idx3_ref, loss_ref, cbp_ref,
                accp_ref, accs_ref):
    i = pl.program_id(0)
    cbt = cbt_ref[...]
    mms = []
    x2s = []
    for b in range(_BPT):
        xb = xt_ref[b]
        mms.append(lax.dot_general(xb, cbt, (((0,), (0,)), ((), ())),
                                   preferred_element_type=jnp.float32))
        x2s.append(jnp.sum(xb * xb, axis=0))
    mm = jnp.concatenate(mms, axis=0)
    x2 = jnp.concatenate(x2s, axis=0).reshape(_TILE, 1)
    c2 = jnp.sum(cbt * cbt, axis=0)[None, :]
    d2 = jnp.maximum(x2 + c2 - 2.0 * mm, 0.0)
    d = jnp.sqrt(d2)

    dmin = jnp.min(d, axis=1, keepdims=True)
    kio = lax.broadcasted_iota(jnp.int32, (_TILE, _K), 1)
    idx = jnp.min(jnp.where(d == dmin, kio, _K), axis=1)
    idx_ref[pl.ds(i * _TILE, _TILE)] = idx
    idx3_ref[...] = idx.reshape(_WPT, _NCHUNK, _CHUNK)

    sh = dmin - d
    p = jnp.exp(sh)
    s = jnp.sum(p, axis=1, keepdims=True)
    t = jnp.sum(p * sh, axis=1, keepdims=True)
    samp = jnp.sum(t / s - jnp.log(s))
    sq = jnp.sum(dmin * dmin)
    pcol = jnp.sum(p * (1.0 / s), axis=0, keepdims=True)

    @pl.when(i == 0)
    def _():
        cbp_ref[:, :_D] = lax.transpose(cbt, (1, 0))
        cbp_ref[:, _D:] = jnp.zeros((_K, _DPAD - _D), jnp.float32)
        accp_ref[...] = jnp.zeros_like(accp_ref)
        accs_ref[0] = 0.0
        accs_ref[1] = 0.0

    accp_ref[...] += pcol
    accs_ref[0] += sq
    accs_ref[1] += samp

    @pl.when(i == _NTILES - 1)
    def _():
        avgp = accp_ref[...] / _N
        avg_ent = -jnp.sum(avgp * jnp.log(avgp + 1e-5))
        sample_ent = -(accs_ref[1] / _N)
        latent = (1.0 + _COMMITMENT_COST) * (accs_ref[0] / (_N * _D))
        loss = latent + _ENTROPY_LOSS_RATIO * (sample_ent - avg_ent)
        loss_ref[...] = jnp.full((1, 128), loss, jnp.float32)


_vq_tc = pl.pallas_call(
    _vq_tc_body,
    grid=(_NTILES,),
    in_specs=[
        pl.BlockSpec((_BPT, _D, _T), lambda i: (i, 0, 0)),
        pl.BlockSpec((_D, _K), lambda i: (0, 0)),
    ],
    out_specs=[
        pl.BlockSpec((_N,), lambda i: (0,)),
        pl.BlockSpec((_WPT, _NCHUNK, _CHUNK), lambda i: (i, 0, 0)),
        pl.BlockSpec((1, 128), lambda i: (0, 0)),
        pl.BlockSpec((_K, _DPAD), lambda i: (0, 0)),
    ],
    out_shape=[
        jax.ShapeDtypeStruct((_N,), jnp.int32),
        jax.ShapeDtypeStruct((_NW, _NCHUNK, _CHUNK), jnp.int32),
        jax.ShapeDtypeStruct((1, 128), jnp.float32),
        jax.ShapeDtypeStruct((_K, _DPAD), jnp.float32),
    ],
    scratch_shapes=[
        pltpu.VMEM((1, _K), jnp.float32),
        pltpu.SMEM((2,), jnp.float32),
    ],
)


def _sc_gather_body(cb_hbm, idx_hbm, out_hbm, idx_v, rows_v,
                    gsem0, gsem1, gsem2, ssem):
    c = lax.axis_index("c")
    s = lax.axis_index("s")
    wid = s * 2 + c
    base = wid * _ROWS_PER_W
    gsems = [gsem0, gsem1, gsem2]
    pltpu.sync_copy(idx_hbm.at[wid], idx_v)
    gathers = [
        pltpu.async_copy(cb_hbm.at[idx_v.at[j]], rows_v.at[j], gsems[j])
        for j in range(_NCHUNK)
    ]
    scatters = []
    for j in range(_NCHUNK):
        gathers[j].wait()
        scatters.append(pltpu.async_copy(
            rows_v.at[j], out_hbm.at[pl.ds(base + j * _CHUNK, _CHUNK)],
            ssem))
    for sc in scatters:
        sc.wait()


_sc_gather_cache = []


def _sc_gather(cb_pad, idx3):
    if not _sc_gather_cache:
        _sc_gather_cache.append(pl.kernel(
            _sc_gather_body,
            out_type=jax.ShapeDtypeStruct((_N, _DPAD), jnp.float32),
            mesh=plsc.VectorSubcoreMesh(core_axis_name="c",
                                        subcore_axis_name="s"),
            scratch_types=[
                pltpu.VMEM((_NCHUNK, _CHUNK), jnp.int32),
                pltpu.VMEM((_NCHUNK, _CHUNK, _DPAD), jnp.float32),
                pltpu.SemaphoreType.DMA,
                pltpu.SemaphoreType.DMA,
                pltpu.SemaphoreType.DMA,
                pltpu.SemaphoreType.DMA,
            ],
        ))
    return _sc_gather_cache[0](cb_pad, idx3)


def kernel(x, codebook):
    xt = jnp.swapaxes(x, 1, 2)
    cbt = codebook.T
    idx, idx3, loss_out, cb_pad = _vq_tc(xt, cbt)
    quantized = _sc_gather(cb_pad, idx3)
    return quantized[:, :_D].reshape(x.shape), loss_out[0, 0], idx
